# baseline (speedup 1.0000x reference)
import functools

import jax
import jax.numpy as jnp
from jax import lax
from jax.experimental import pallas as pl
from jax.experimental.pallas import tpu as pltpu
from jax.experimental.pallas import tpu_sc as plsc

VOCAB = 100000
D = 128
B = 1024
S = 200
N = B * S
EPS = 1e-6

NUM_CORES = 2
NUM_SUBCORES = 16
NW = NUM_CORES * NUM_SUBCORES
ROWS_PER_W = N // NW
CHUNK = 128
NCHUNK = ROWS_PER_W // CHUNK
NG = CHUNK // 16


def _vec_sqrt(v):
    v = jnp.maximum(v, 0.0)
    i = lax.bitcast_convert_type(v, jnp.int32)
    g = lax.bitcast_convert_type(
        lax.shift_right_logical(i, 1) + jnp.int32(0x1FBD1DF5), jnp.float32)
    g = 0.5 * (g + v / g)
    g = 0.5 * (g + v / g)
    g = 0.5 * (g + v / g)
    return g


def _stats_to_scale(acc, acc2):
    mean = acc * (1.0 / D)
    var = (acc2 - acc * mean) * (1.0 / (D - 1))
    return mean, 1.0 / (_vec_sqrt(var) + EPS)


def _normalize_chunk(buf_v, rvecs, means, invs, gam_v, bet_v):
    def norm_col(j, _):
        jv = jnp.full((16,), j, dtype=jnp.int32)
        z = jnp.zeros((16,), dtype=jnp.int32)
        gs = plsc.load_gather(gam_v, [z, jv])
        bs = plsc.load_gather(bet_v, [z, jv])
        for g in range(NG):
            v = plsc.load_gather(buf_v, [rvecs[g], jv])
            y = (v - means[g]) * invs[g] * gs + bs
            plsc.store_scatter(buf_v, [rvecs[g], jv], y)
        return 0

    lax.fori_loop(0, D, norm_col, 0)


def _fused(src0_flat, src1_flat, seg_flat, W0, W1, pos_slice, seg_table,
           gamma0, beta0, gamma1, beta1):
    mesh = plsc.VectorSubcoreMesh(core_axis_name="c", subcore_axis_name="s")

    @functools.partial(
        pl.kernel,
        mesh=mesh,
        compiler_params=pltpu.CompilerParams(needs_layout_passes=False),
        out_type=[
            jax.ShapeDtypeStruct((N, D), jnp.float32),
            jax.ShapeDtypeStruct((N, D), jnp.float32),
        ],
        scratch_types=[
            pltpu.VMEM((ROWS_PER_W,), jnp.int32),
            pltpu.VMEM((ROWS_PER_W,), jnp.int32),
            pltpu.VMEM((S, D), jnp.float32),
            pltpu.VMEM((3, D), jnp.float32),
            pltpu.VMEM((1, D), jnp.float32),
            pltpu.VMEM((1, D), jnp.float32),
            pltpu.VMEM((CHUNK, D), jnp.float32),
            pltpu.SemaphoreType.DMA,
        ],
    )
    def body(w0_hbm, w1_hbm, i0_hbm, i1_hbm, sg_hbm, pos_hbm, st_hbm,
             g0_hbm, b0_hbm, g1_hbm, b1_hbm, o0_hbm, o1_hbm,
             idx_v, seg_v, pos_v, st_v, gam_v, bet_v, buf_v, sem):
        wid = lax.axis_index("s") * NUM_CORES + lax.axis_index("c")
        base = wid * ROWS_PER_W
        iota = lax.iota(jnp.int32, 16)
        rvecs = [iota + 16 * g for g in range(NG)]
        zero = jnp.zeros((16,), jnp.float32)

        pltpu.sync_copy(i0_hbm.at[pl.ds(base, ROWS_PER_W)], idx_v)
        pltpu.sync_copy(g0_hbm, gam_v)
        pltpu.sync_copy(b0_hbm, bet_v)

        def step0(i, _):
            off = i * CHUNK
            pltpu.async_copy(
                w0_hbm.at[idx_v.at[pl.ds(off, CHUNK)]], buf_v, sem).wait()

            def stats_col(j, carry):
                jv = jnp.full((16,), j, dtype=jnp.int32)
                out = []
                for g in range(NG):
                    v = plsc.load_gather(buf_v, [rvecs[g], jv])
                    out.append(carry[2 * g] + v)
                    out.append(carry[2 * g + 1] + v * v)
                return tuple(out)

            accs = lax.fori_loop(
                0, D, stats_col, tuple(zero for _ in range(2 * NG)))
            means = []
            invs = []
            for g in range(NG):
                m, iv = _stats_to_scale(accs[2 * g], accs[2 * g + 1])
                means.append(m)
                invs.append(iv)
            _normalize_chunk(buf_v, rvecs, means, invs, gam_v, bet_v)
            pltpu.sync_copy(buf_v, o0_hbm.at[pl.ds(base + off, CHUNK)])
            return 0

        lax.fori_loop(0, NCHUNK, step0, 0)

        pltpu.sync_copy(i1_hbm.at[pl.ds(base, ROWS_PER_W)], idx_v)
        pltpu.sync_copy(sg_hbm.at[pl.ds(base, ROWS_PER_W)], seg_v)
        pltpu.sync_copy(pos_hbm, pos_v)
        pltpu.sync_copy(st_hbm, st_v)
        pltpu.sync_copy(g1_hbm, gam_v)
        pltpu.sync_copy(b1_hbm, bet_v)

        def step1(i, _):
            off = i * CHUNK
            pltpu.async_copy(
                w1_hbm.at[idx_v.at[pl.ds(off, CHUNK)]], buf_v, sem).wait()
            pvecs = [
                lax.rem(base + off + 16 * g + iota, jnp.int32(S))
                for g in range(NG)
            ]
            svecs = [seg_v[pl.ds(off + 16 * g, 16)] for g in range(NG)]

            def stats_col(j, carry):
                jv = jnp.full((16,), j, dtype=jnp.int32)
                out = []
                for g in range(NG):
                    w = plsc.load_gather(buf_v, [rvecs[g], jv])
                    p = plsc.load_gather(pos_v, [pvecs[g], jv])
                    sg = plsc.load_gather(st_v, [svecs[g], jv])
                    x = w + p + sg
                    plsc.store_scatter(buf_v, [rvecs[g], jv], x)
                    out.append(carry[2 * g] + x)
                    out.append(carry[2 * g + 1] + x * x)
                return tuple(out)

            accs = lax.fori_loop(
                0, D, stats_col, tuple(zero for _ in range(2 * NG)))
            means = []
            invs = []
            for g in range(NG):
                m, iv = _stats_to_scale(accs[2 * g], accs[2 * g + 1])
                means.append(m)
                invs.append(iv)
            _normalize_chunk(buf_v, rvecs, means, invs, gam_v, bet_v)
            pltpu.sync_copy(buf_v, o1_hbm.at[pl.ds(base + off, CHUNK)])
            return 0

        lax.fori_loop(0, NCHUNK, step1, 0)

    return body(W0, W1, src0_flat, src1_flat, seg_flat, pos_slice, seg_table,
                gamma0, beta0, gamma1, beta1)


def kernel(src_0, src_1, seg_0, seg_1, W0, gamma0, beta0, W1, pos_table,
           seg_table, gamma1, beta1):
    src0_flat = src_0.reshape(N).astype(jnp.int32)
    src1_flat = src_1.reshape(N).astype(jnp.int32)
    seg_flat = seg_1.reshape(N).astype(jnp.int32)
    e0, e1 = _fused(
        src0_flat, src1_flat, seg_flat, W0, W1, pos_table[:S], seg_table,
        gamma0.reshape(1, D), beta0.reshape(1, D),
        gamma1.reshape(1, D), beta1.reshape(1, D))
    return (e0.reshape(B, S, D), e1.reshape(B, S, D))


# speedup vs baseline: 13.4460x; 13.4460x over previous
import functools

import jax
import jax.numpy as jnp
from jax import lax
from jax.experimental import pallas as pl
from jax.experimental.pallas import tpu as pltpu
from jax.experimental.pallas import tpu_sc as plsc

VOCAB = 100000
D = 128
B = 1024
S = 200
N = B * S
EPS = 1e-6

NUM_CORES = 2
NUM_SUBCORES = 16
NW = NUM_CORES * NUM_SUBCORES
ROWS_PER_W = N // NW
CHUNK = 128
NCHUNK = ROWS_PER_W // CHUNK


def _dual_gather(src0_flat, src1_flat, W0, W1):
    mesh = plsc.VectorSubcoreMesh(core_axis_name="c", subcore_axis_name="s")

    @functools.partial(
        pl.kernel,
        mesh=mesh,
        out_type=[
            jax.ShapeDtypeStruct((N, D), jnp.float32),
            jax.ShapeDtypeStruct((N, D), jnp.float32),
        ],
        scratch_types=[
            pltpu.VMEM((ROWS_PER_W,), jnp.int32),
            pltpu.VMEM((ROWS_PER_W,), jnp.int32),
            pltpu.VMEM((CHUNK, D), jnp.float32),
            pltpu.VMEM((CHUNK, D), jnp.float32),
            pltpu.VMEM((CHUNK, D), jnp.float32),
            pltpu.VMEM((CHUNK, D), jnp.float32),
            pltpu.SemaphoreType.DMA,
            pltpu.SemaphoreType.DMA,
            pltpu.SemaphoreType.DMA,
            pltpu.SemaphoreType.DMA,
            pltpu.SemaphoreType.DMA,
            pltpu.SemaphoreType.DMA,
            pltpu.SemaphoreType.DMA,
            pltpu.SemaphoreType.DMA,
        ],
    )
    def body(w0_hbm, w1_hbm, i0_hbm, i1_hbm, o0_hbm, o1_hbm,
             idx0_v, idx1_v, b00, b01, b10, b11,
             gs00, gs01, gs10, gs11, os00, os01, os10, os11):
        wid = lax.axis_index("s") * NUM_CORES + lax.axis_index("c")
        base = wid * ROWS_PER_W
        pltpu.sync_copy(i0_hbm.at[pl.ds(base, ROWS_PER_W)], idx0_v)
        pltpu.sync_copy(i1_hbm.at[pl.ds(base, ROWS_PER_W)], idx1_v)

        streams = (
            (w0_hbm, idx0_v, o0_hbm, (b00, b01), (gs00, gs01), (os00, os01)),
            (w1_hbm, idx1_v, o1_hbm, (b10, b11), (gs10, gs11), (os10, os11)),
        )

        def startg(w, idx, buf, gsem, i):
            pltpu.async_copy(w.at[idx.at[pl.ds(i * CHUNK, CHUNK)]], buf, gsem)

        def waitg(w, buf, gsem):
            pltpu.make_async_copy(w.at[pl.ds(0, CHUNK)], buf, gsem).wait()

        def starto(o, buf, osem, i):
            pltpu.async_copy(buf, o.at[pl.ds(base + i * CHUNK, CHUNK)], osem)

        def waito(o, buf, osem):
            pltpu.make_async_copy(buf, o.at[pl.ds(0, CHUNK)], osem).wait()

        for w, idx, o, bufs, gsems, osems in streams:
            startg(w, idx, bufs[0], gsems[0], 0)
            startg(w, idx, bufs[1], gsems[1], 1)

        def step(k, _):
            for b in range(2):
                i = 2 * k + b
                for w, idx, o, bufs, gsems, osems in streams:
                    waitg(w, bufs[b], gsems[b])
                    starto(o, bufs[b], osems[b], i)
                for w, idx, o, bufs, gsems, osems in streams:
                    waito(o, bufs[b], osems[b])

                    @pl.when(i + 2 < NCHUNK)
                    def _():
                        startg(w, idx, bufs[b], gsems[b], i + 2)
            return 0

        lax.fori_loop(0, NCHUNK // 2, step, 0)

    return body(W0, W1, src0_flat, src1_flat)


BB = 16


def _ln_kernel(raw0_ref, raw1_ref, seg_ref, pos_ref, segtab_ref,
               g0_ref, b0_ref, g1_ref, b1_ref, o0_ref, o1_ref):
    g0 = g0_ref[...]
    b0 = b0_ref[...]
    g1 = g1_ref[...]
    b1 = b1_ref[...]

    def ln(x, g, bta):
        mean = jnp.mean(x, axis=-1, keepdims=True)
        xm = x - mean
        var = jnp.sum(xm * xm, axis=-1, keepdims=True) * (1.0 / (D - 1))
        inv = 1.0 / (jnp.sqrt(var) + EPS)
        return xm * inv * g + bta

    x0 = raw0_ref[...]
    o0_ref[...] = ln(x0, g0, b0)

    seg = seg_ref[...][..., None]
    st = segtab_ref[...]
    segemb = jnp.where(seg == 0, st[0], jnp.where(seg == 1, st[1], st[2]))
    x1 = raw1_ref[...] + pos_ref[...][None, :, :] + segemb
    o1_ref[...] = ln(x1, g1, b1)


def _ln_call(raw0, raw1, seg_1, pos_slice, seg_table, gamma0, beta0, gamma1, beta1):
    grid = (B // BB,)
    return pl.pallas_call(
        _ln_kernel,
        grid=grid,
        in_specs=[
            pl.BlockSpec((BB, S, D), lambda i: (i, 0, 0)),
            pl.BlockSpec((BB, S, D), lambda i: (i, 0, 0)),
            pl.BlockSpec((BB, S), lambda i: (i, 0)),
            pl.BlockSpec((S, D), lambda i: (0, 0)),
            pl.BlockSpec((3, D), lambda i: (0, 0)),
            pl.BlockSpec((1, D), lambda i: (0, 0)),
            pl.BlockSpec((1, D), lambda i: (0, 0)),
            pl.BlockSpec((1, D), lambda i: (0, 0)),
            pl.BlockSpec((1, D), lambda i: (0, 0)),
        ],
        out_specs=[
            pl.BlockSpec((BB, S, D), lambda i: (i, 0, 0)),
            pl.BlockSpec((BB, S, D), lambda i: (i, 0, 0)),
        ],
        out_shape=[
            jax.ShapeDtypeStruct((B, S, D), jnp.float32),
            jax.ShapeDtypeStruct((B, S, D), jnp.float32),
        ],
    )(raw0, raw1, seg_1, pos_slice, seg_table, gamma0, beta0, gamma1, beta1)


def kernel(src_0, src_1, seg_0, seg_1, W0, gamma0, beta0, W1, pos_table,
           seg_table, gamma1, beta1):
    src0_flat = src_0.reshape(N).astype(jnp.int32)
    src1_flat = src_1.reshape(N).astype(jnp.int32)
    raw0, raw1 = _dual_gather(src0_flat, src1_flat, W0, W1)
    raw0 = raw0.reshape(B, S, D)
    raw1 = raw1.reshape(B, S, D)
    e0, e1 = _ln_call(
        raw0, raw1, seg_1.astype(jnp.int32), pos_table[:S], seg_table,
        gamma0.reshape(1, D), beta0.reshape(1, D),
        gamma1.reshape(1, D), beta1.reshape(1, D))
    return (e0, e1)
